# baseline (speedup 1.0000x reference)
import functools

import jax
import jax.numpy as jnp
from jax import lax
from jax.experimental import pallas as pl
from jax.experimental.pallas import tpu as pltpu
from jax.experimental.pallas import tpu_sc as plsc

_DIM = 64
_CODES = 16

_mesh = plsc.ScalarSubcoreMesh(axis_name="c", num_cores=1)


@functools.partial(
    pl.kernel,
    out_type=(
        jax.ShapeDtypeStruct((1, _CODES), jnp.float32),
        jax.ShapeDtypeStruct((1, 1, _DIM), jnp.float32),
    ),
    mesh=_mesh,
    compiler_params=pltpu.CompilerParams(
        needs_layout_passes=False,
        disable_bounds_checks=True,
    ),
    scratch_types=[
        pltpu.SMEM((1, _DIM), jnp.float32),
        pltpu.SMEM((1, _CODES, _DIM), jnp.float32),
        pltpu.SMEM((1, _CODES), jnp.float32),
        pltpu.SMEM((1, 1, _DIM), jnp.float32),
        pltpu.SemaphoreType.DMA,
        pltpu.SemaphoreType.DMA,
    ],
)
def _vq_kernel(x_hbm, cb_hbm, onehot_hbm, resid_hbm, x_s, cb_s, oh_s, r_s,
               sem_a, sem_b):
    in_a = pltpu.async_copy(cb_hbm, cb_s, sem_a)
    in_b = pltpu.async_copy(x_hbm, x_s, sem_b)
    in_a.wait()
    in_b.wait()

    def code_body(c, carry):
        best, best_idx = carry
        s = jnp.float32(0.0)
        for d in range(_DIM):
            t = x_s[0, d] - cb_s[0, c, d]
            s = s + t * t
        take = s < best
        return (jnp.where(take, s, best), jnp.where(take, c, best_idx))

    best, best_idx = lax.fori_loop(
        0, _CODES, code_body, (jnp.float32(3.4e38), jnp.int32(0))
    )
    for j in range(_CODES):
        oh_s[0, j] = jnp.where(best_idx == j, 1.0, 0.0).astype(jnp.float32)

    def resid_body(d, _):
        r_s[0, 0, d] = x_s[0, d] - cb_s[0, best_idx, d]
        return 0

    lax.fori_loop(0, _DIM, resid_body, 0)
    out_a = pltpu.async_copy(oh_s, onehot_hbm, sem_a)
    out_b = pltpu.async_copy(r_s, resid_hbm, sem_b)
    out_a.wait()
    out_b.wait()


def kernel(inputs, codebook):
    return _vq_kernel(inputs, codebook)


# speedup vs baseline: 1.0911x; 1.0911x over previous
import functools

import jax
import jax.numpy as jnp
from jax import lax
from jax.experimental import pallas as pl
from jax.experimental.pallas import tpu as pltpu
from jax.experimental.pallas import tpu_sc as plsc

_LANES = 16
_DIM = 64
_CODES = 16
_UNROLL = 4

_mesh = plsc.VectorSubcoreMesh(
    core_axis_name="c", subcore_axis_name="s", num_cores=1, num_subcores=1
)


@functools.partial(
    pl.kernel,
    out_type=(
        jax.ShapeDtypeStruct((1, _CODES), jnp.float32),
        jax.ShapeDtypeStruct((1, 1, _DIM), jnp.float32),
    ),
    mesh=_mesh,
    compiler_params=pltpu.CompilerParams(
        needs_layout_passes=False,
        disable_bounds_checks=True,
    ),
    scratch_types=[
        pltpu.VMEM((1, _DIM), jnp.float32),
        pltpu.VMEM((1, _CODES, _DIM), jnp.float32),
        pltpu.VMEM((1, _CODES), jnp.float32),
        pltpu.VMEM((1, 1, _DIM), jnp.float32),
        pltpu.SemaphoreType.DMA,
        pltpu.SemaphoreType.DMA,
    ],
)
def _vq_kernel(x_hbm, cb_hbm, onehot_hbm, resid_hbm, x_v, cb_v, oh_v, r_v,
               sem_a, sem_b):
    @pl.when(lax.axis_index("s") == 0)
    def _():
        in_a = pltpu.async_copy(cb_hbm, cb_v, sem_a)
        in_b = pltpu.async_copy(x_hbm, x_v, sem_b)
        in_a.wait()
        in_b.wait()
        lanes = lax.iota(jnp.int32, _LANES)
        zero = jnp.zeros((_LANES,), jnp.int32)

        def dist_body(i, accs):
            base = i * _UNROLL
            out = []
            for k in range(_UNROLL):
                d_splat = jnp.broadcast_to(base + k, (_LANES,))
                col = plsc.load_gather(cb_v, [zero, lanes, d_splat])
                xb = plsc.load_gather(x_v, [zero, d_splat])
                t = xb - col
                out.append(accs[k] + t * t)
            return tuple(out)

        z = jnp.zeros((_LANES,), jnp.float32)
        acc = lax.fori_loop(0, _DIM // _UNROLL, dist_body, (z, z, z, z))
        dist = (acc[0] + acc[1]) + (acc[2] + acc[3])
        m = jnp.min(dist)
        idx = plsc.all_reduce_ffs(dist == m)
        oh_v[0, :] = jnp.where(lanes == idx, 1.0, 0.0).astype(jnp.float32)
        out_a = pltpu.async_copy(oh_v, onehot_hbm, sem_a)
        for i in range(_DIM // _LANES):
            xi = x_v[0, pl.ds(_LANES * i, _LANES)]
            row = plsc.load_gather(cb_v, [zero, idx, lanes + _LANES * i])
            r_v[0, 0, pl.ds(_LANES * i, _LANES)] = xi - row
        out_b = pltpu.async_copy(r_v, resid_hbm, sem_b)
        out_a.wait()
        out_b.wait()


def kernel(inputs, codebook):
    return _vq_kernel(inputs, codebook)
